# baseline (speedup 1.0000x reference)
import functools

import jax
import jax.numpy as jnp
from jax import lax
from jax.experimental import pallas as pl
from jax.experimental.pallas import tpu as pltpu
from jax.experimental.pallas import tpu_sc as plsc

N_NODES = 10000
N_EDGES = 320000
D = 128
NC = 2
NS = 16
NW = NC * NS
XCH = 128
NCHUNK = N_EDGES // XCH
NX = NCHUNK - (NCHUNK // NW) * NW
EPT = (NCHUNK // NW) * XCH
IDXW = EPT + XCH
CH = 128
CPT = EPT // CH
ZT = 10
ZROWS = N_NODES // ZT

_mesh = plsc.VectorSubcoreMesh(core_axis_name="c", subcore_axis_name="s")


@functools.partial(
    pl.kernel,
    out_type=jax.ShapeDtypeStruct((NC, N_NODES, D), jnp.float32),
    mesh=_mesh,
    scratch_types=[
        pltpu.VMEM((IDXW,), jnp.int32),
        pltpu.VMEM((2, CH, D), jnp.float32),
        pltpu.VMEM((16, D), jnp.float32),
        pltpu.VMEM_SHARED((N_NODES, D), jnp.float32),
        pltpu.SemaphoreType.DMA,
        pltpu.SemaphoreType.DMA,
    ],
)
def _scatter_sc(ei_hbm, ea_hbm, out_hbm,
                idx_v, ebuf, zbuf, agg_s, rs0, rs1):
    cid = lax.axis_index("c")
    sid = lax.axis_index("s")
    wid = sid * NC + cid
    ebase = wid * EPT

    pltpu.async_copy(ei_hbm.at[0, pl.ds(ebase, EPT)],
                     idx_v.at[pl.ds(0, EPT)], rs1)
    z16 = jnp.zeros((16,), jnp.float32)
    for i in range(16):
        zrow = zbuf.at[i]
        for k in range(D // 16):
            zrow[pl.ds(16 * k, 16)] = z16

    @pl.when(sid < ZT)
    def _():
        def zstep(i, carry):
            pltpu.async_copy(zbuf,
                             agg_s.at[pl.ds(sid * ZROWS + 16 * i, 16)], rs0)
            return carry

        lax.fori_loop(0, ZROWS // 16, zstep, 0)
        zrem = ZROWS - (ZROWS // 16) * 16
        if zrem:
            pltpu.async_copy(
                zbuf.at[pl.ds(0, zrem)],
                agg_s.at[pl.ds(sid * ZROWS + ZROWS - zrem, zrem)], rs0)

        def zdrain(i, carry):
            pltpu.make_async_copy(
                zbuf, agg_s.at[pl.ds(sid * ZROWS, 16)], rs0).wait()
            return carry

        lax.fori_loop(0, ZROWS // 16, zdrain, 0)
        if zrem:
            pltpu.make_async_copy(
                zbuf.at[pl.ds(0, zrem)],
                agg_s.at[pl.ds(sid * ZROWS, zrem)], rs0).wait()

    pltpu.make_async_copy(ei_hbm.at[0, pl.ds(ebase, EPT)],
                          idx_v.at[pl.ds(0, EPT)], rs1).wait()

    @pl.when(wid < NX)
    def _():
        pltpu.sync_copy(ei_hbm.at[0, pl.ds(NW * EPT + wid * XCH, XCH)],
                        idx_v.at[pl.ds(EPT, XCH)])

    plsc.subcore_barrier()

    buf0 = ebuf.at[0]
    buf1 = ebuf.at[1]
    pltpu.async_copy(ea_hbm.at[pl.ds(ebase, CH)], buf0, rs0)

    def step(i, carry):
        j0 = 2 * i
        j1 = 2 * i + 1
        pltpu.async_copy(ea_hbm.at[pl.ds(ebase + j1 * CH, CH)], buf1, rs1)
        pltpu.make_async_copy(ea_hbm.at[pl.ds(ebase, CH)], buf0, rs0).wait()
        pltpu.sync_copy(buf0, agg_s.at[idx_v.at[pl.ds(j0 * CH, CH)]],
                        add=True)
        j2 = jnp.minimum(j0 + 2, CPT - 1)
        pltpu.async_copy(ea_hbm.at[pl.ds(ebase + j2 * CH, CH)], buf0, rs0)
        pltpu.make_async_copy(ea_hbm.at[pl.ds(ebase, CH)], buf1, rs1).wait()
        pltpu.sync_copy(buf1, agg_s.at[idx_v.at[pl.ds(j1 * CH, CH)]],
                        add=True)
        return carry

    lax.fori_loop(0, CPT // 2, step, 0)
    pltpu.make_async_copy(ea_hbm.at[pl.ds(ebase, CH)], buf0, rs0).wait()

    @pl.when(wid < NX)
    def _():
        pltpu.sync_copy(ea_hbm.at[pl.ds(NW * EPT + wid * XCH, XCH)], buf0)
        pltpu.sync_copy(buf0, agg_s.at[idx_v.at[pl.ds(EPT, XCH)]], add=True)

    plsc.subcore_barrier()

    @pl.when(sid < ZT)
    def _():
        pltpu.sync_copy(agg_s.at[pl.ds(sid * ZROWS, ZROWS)],
                        out_hbm.at[cid, pl.ds(sid * ZROWS, ZROWS)])


BN = 1000


def _mlp_a_body(x_ref, w1x_ref, b1_ref, t_ref):
    t_ref[...] = (jnp.dot(x_ref[...], w1x_ref[...],
                          preferred_element_type=jnp.float32) + b1_ref[...])


def _mlp_a(x, w1x, b1):
    return pl.pallas_call(
        _mlp_a_body,
        grid=(N_NODES // BN,),
        in_specs=[
            pl.BlockSpec((BN, D), lambda i: (i, 0)),
            pl.BlockSpec((D, D), lambda i: (0, 0)),
            pl.BlockSpec((1, D), lambda i: (0, 0)),
        ],
        out_specs=pl.BlockSpec((BN, D), lambda i: (i, 0)),
        out_shape=jax.ShapeDtypeStruct((N_NODES, D), jnp.float32),
    )(x, w1x, b1)


def _mlp_b_body(t_ref, p_ref, w1a_ref, w2_ref, b2_ref, o_ref):
    agg = p_ref[0] + p_ref[1]
    h = t_ref[...] + jnp.dot(agg, w1a_ref[...],
                             preferred_element_type=jnp.float32)
    h = jnp.maximum(h, 0.0)
    o_ref[...] = (jnp.dot(h, w2_ref[...], preferred_element_type=jnp.float32)
                  + b2_ref[...])


def _mlp_b(t, parts, w1a, w2, b2):
    return pl.pallas_call(
        _mlp_b_body,
        grid=(N_NODES // BN,),
        in_specs=[
            pl.BlockSpec((BN, D), lambda i: (i, 0)),
            pl.BlockSpec((NC, BN, D), lambda i: (0, i, 0)),
            pl.BlockSpec((D, D), lambda i: (0, 0)),
            pl.BlockSpec((D, D), lambda i: (0, 0)),
            pl.BlockSpec((1, D), lambda i: (0, 0)),
        ],
        out_specs=pl.BlockSpec((BN, D), lambda i: (i, 0)),
        out_shape=jax.ShapeDtypeStruct((N_NODES, D), jnp.float32),
    )(t, parts, w1a, w2, b2)


def kernel(x, edge_index, edge_attr, u, batch, W1, b1, W2, b2):
    parts = _scatter_sc(edge_index.astype(jnp.int32), edge_attr)
    w1T = W1.T
    t = _mlp_a(x, w1T[:D], b1.reshape(1, D))
    return _mlp_b(t, parts, w1T[D:], W2.T, b2.reshape(1, D))


# speedup vs baseline: 1.0268x; 1.0268x over previous
import functools

import jax
import jax.numpy as jnp
from jax import lax
from jax.experimental import pallas as pl
from jax.experimental.pallas import tpu as pltpu
from jax.experimental.pallas import tpu_sc as plsc

N_NODES = 10000
N_EDGES = 320000
D = 128
NC = 2
NS = 16
NW = NC * NS
XCH = 128
NCHUNK = N_EDGES // XCH
NX = NCHUNK - (NCHUNK // NW) * NW
EPT = (NCHUNK // NW) * XCH
IDXW = EPT + XCH
CH = 128
CPT = EPT // CH
ZT = 10
ZROWS = N_NODES // ZT

_mesh = plsc.VectorSubcoreMesh(core_axis_name="c", subcore_axis_name="s")


@functools.partial(
    pl.kernel,
    out_type=jax.ShapeDtypeStruct((NC, N_NODES, D), jnp.float32),
    mesh=_mesh,
    scratch_types=[
        pltpu.VMEM((IDXW,), jnp.int32),
        pltpu.VMEM((2, CH, D), jnp.float32),
        pltpu.VMEM((16, D), jnp.float32),
        pltpu.VMEM_SHARED((N_NODES, D), jnp.float32),
        pltpu.SemaphoreType.DMA,
        pltpu.SemaphoreType.DMA,
    ],
)
def _scatter_sc(ei_hbm, ea_hbm, out_hbm,
                idx_v, ebuf, zbuf, agg_s, rs0, rs1):
    cid = lax.axis_index("c")
    sid = lax.axis_index("s")
    wid = sid * NC + cid
    ebase = wid * EPT

    pltpu.async_copy(ei_hbm.at[0, pl.ds(ebase, EPT)],
                     idx_v.at[pl.ds(0, EPT)], rs1)
    z16 = jnp.zeros((16,), jnp.float32)
    for i in range(16):
        zrow = zbuf.at[i]
        for k in range(D // 16):
            zrow[pl.ds(16 * k, 16)] = z16

    @pl.when(sid < ZT)
    def _():
        def zstep(i, carry):
            pltpu.async_copy(zbuf,
                             agg_s.at[pl.ds(sid * ZROWS + 16 * i, 16)], rs0)
            return carry

        lax.fori_loop(0, ZROWS // 16, zstep, 0)
        zrem = ZROWS - (ZROWS // 16) * 16
        if zrem:
            pltpu.async_copy(
                zbuf.at[pl.ds(0, zrem)],
                agg_s.at[pl.ds(sid * ZROWS + ZROWS - zrem, zrem)], rs0)

        def zdrain(i, carry):
            pltpu.make_async_copy(
                zbuf, agg_s.at[pl.ds(sid * ZROWS, 16)], rs0).wait()
            return carry

        lax.fori_loop(0, ZROWS // 16, zdrain, 0)
        if zrem:
            pltpu.make_async_copy(
                zbuf.at[pl.ds(0, zrem)],
                agg_s.at[pl.ds(sid * ZROWS, zrem)], rs0).wait()

    pltpu.make_async_copy(ei_hbm.at[0, pl.ds(ebase, EPT)],
                          idx_v.at[pl.ds(0, EPT)], rs1).wait()

    @pl.when(wid < NX)
    def _():
        pltpu.sync_copy(ei_hbm.at[0, pl.ds(NW * EPT + wid * XCH, XCH)],
                        idx_v.at[pl.ds(EPT, XCH)])

    plsc.subcore_barrier()

    buf0 = ebuf.at[0]
    buf1 = ebuf.at[1]
    pltpu.async_copy(ea_hbm.at[pl.ds(ebase, CH)], buf0, rs0)

    def step(i, carry):
        j0 = 2 * i
        j1 = 2 * i + 1
        pltpu.async_copy(ea_hbm.at[pl.ds(ebase + j1 * CH, CH)], buf1, rs1)
        pltpu.make_async_copy(ea_hbm.at[pl.ds(ebase, CH)], buf0, rs0).wait()
        pltpu.sync_copy(buf0, agg_s.at[idx_v.at[pl.ds(j0 * CH, CH)]],
                        add=True)
        j2 = jnp.minimum(j0 + 2, CPT - 1)
        pltpu.async_copy(ea_hbm.at[pl.ds(ebase + j2 * CH, CH)], buf0, rs0)
        pltpu.make_async_copy(ea_hbm.at[pl.ds(ebase, CH)], buf1, rs1).wait()
        pltpu.sync_copy(buf1, agg_s.at[idx_v.at[pl.ds(j1 * CH, CH)]],
                        add=True)
        return carry

    lax.fori_loop(0, CPT // 2, step, 0)
    pltpu.make_async_copy(ea_hbm.at[pl.ds(ebase, CH)], buf0, rs0).wait()

    @pl.when(wid < NX)
    def _():
        pltpu.sync_copy(ea_hbm.at[pl.ds(NW * EPT + wid * XCH, XCH)], buf0)
        pltpu.sync_copy(buf0, agg_s.at[idx_v.at[pl.ds(EPT, XCH)]], add=True)

    plsc.subcore_barrier()

    @pl.when(sid < ZT)
    def _():
        pltpu.sync_copy(agg_s.at[pl.ds(sid * ZROWS, ZROWS)],
                        out_hbm.at[cid, pl.ds(sid * ZROWS, ZROWS)])


BN = 2000


def _mlp_a_body(x_ref, w1x_ref, b1_ref, t_ref):
    t_ref[...] = (jnp.dot(x_ref[...], w1x_ref[...],
                          preferred_element_type=jnp.float32) + b1_ref[...])


def _mlp_a(x, w1x, b1):
    return pl.pallas_call(
        _mlp_a_body,
        grid=(N_NODES // BN,),
        in_specs=[
            pl.BlockSpec((BN, D), lambda i: (i, 0)),
            pl.BlockSpec((D, D), lambda i: (0, 0)),
            pl.BlockSpec((1, D), lambda i: (0, 0)),
        ],
        out_specs=pl.BlockSpec((BN, D), lambda i: (i, 0)),
        out_shape=jax.ShapeDtypeStruct((N_NODES, D), jnp.float32),
    )(x, w1x, b1)


def _mlp_b_body(t_ref, p_ref, w1a_ref, w2_ref, b2_ref, o_ref):
    agg = p_ref[0] + p_ref[1]
    h = t_ref[...] + jnp.dot(agg, w1a_ref[...],
                             preferred_element_type=jnp.float32)
    h = jnp.maximum(h, 0.0)
    o_ref[...] = (jnp.dot(h, w2_ref[...], preferred_element_type=jnp.float32)
                  + b2_ref[...])


def _mlp_b(t, parts, w1a, w2, b2):
    return pl.pallas_call(
        _mlp_b_body,
        grid=(N_NODES // BN,),
        in_specs=[
            pl.BlockSpec((BN, D), lambda i: (i, 0)),
            pl.BlockSpec((NC, BN, D), lambda i: (0, i, 0)),
            pl.BlockSpec((D, D), lambda i: (0, 0)),
            pl.BlockSpec((D, D), lambda i: (0, 0)),
            pl.BlockSpec((1, D), lambda i: (0, 0)),
        ],
        out_specs=pl.BlockSpec((BN, D), lambda i: (i, 0)),
        out_shape=jax.ShapeDtypeStruct((N_NODES, D), jnp.float32),
    )(t, parts, w1a, w2, b2)


def kernel(x, edge_index, edge_attr, u, batch, W1, b1, W2, b2):
    parts = _scatter_sc(edge_index.astype(jnp.int32), edge_attr)
    w1T = W1.T
    t = _mlp_a(x, w1T[:D], b1.reshape(1, D))
    return _mlp_b(t, parts, w1T[D:], W2.T, b2.reshape(1, D))


# speedup vs baseline: 1.0328x; 1.0058x over previous
import functools

import jax
import jax.numpy as jnp
from jax import lax
from jax.experimental import pallas as pl
from jax.experimental.pallas import tpu as pltpu
from jax.experimental.pallas import tpu_sc as plsc

N_NODES = 10000
N_EDGES = 320000
D = 128
NC = 2
NS = 16
NW = NC * NS
XCH = 128
NCHUNK = N_EDGES // XCH
NX = NCHUNK - (NCHUNK // NW) * NW
EPT = (NCHUNK // NW) * XCH
IDXW = EPT + XCH
CH = 128
CPT = EPT // CH
ZT = 10
ZROWS = N_NODES // ZT

_mesh = plsc.VectorSubcoreMesh(core_axis_name="c", subcore_axis_name="s")


@functools.partial(
    pl.kernel,
    out_type=jax.ShapeDtypeStruct((NC, N_NODES, D), jnp.float32),
    mesh=_mesh,
    scratch_types=[
        pltpu.VMEM((IDXW,), jnp.int32),
        pltpu.VMEM((2, CH, D), jnp.float32),
        pltpu.VMEM((16, D), jnp.float32),
        pltpu.VMEM_SHARED((N_NODES, D), jnp.float32),
        pltpu.SemaphoreType.DMA,
        pltpu.SemaphoreType.DMA,
    ],
)
def _scatter_sc(ei_hbm, ea_hbm, out_hbm,
                idx_v, ebuf, zbuf, agg_s, rs0, rs1):
    cid = lax.axis_index("c")
    sid = lax.axis_index("s")
    wid = sid * NC + cid
    ebase = wid * EPT

    pltpu.async_copy(ei_hbm.at[0, pl.ds(ebase, EPT)],
                     idx_v.at[pl.ds(0, EPT)], rs1)
    z16 = jnp.zeros((16,), jnp.float32)
    for i in range(16):
        zrow = zbuf.at[i]
        for k in range(D // 16):
            zrow[pl.ds(16 * k, 16)] = z16

    @pl.when(sid < ZT)
    def _():
        def zstep(i, carry):
            pltpu.async_copy(zbuf,
                             agg_s.at[pl.ds(sid * ZROWS + 16 * i, 16)], rs0)
            return carry

        lax.fori_loop(0, ZROWS // 16, zstep, 0)
        zrem = ZROWS - (ZROWS // 16) * 16
        if zrem:
            pltpu.async_copy(
                zbuf.at[pl.ds(0, zrem)],
                agg_s.at[pl.ds(sid * ZROWS + ZROWS - zrem, zrem)], rs0)

        def zdrain(i, carry):
            pltpu.make_async_copy(
                zbuf, agg_s.at[pl.ds(sid * ZROWS, 16)], rs0).wait()
            return carry

        lax.fori_loop(0, ZROWS // 16, zdrain, 0)
        if zrem:
            pltpu.make_async_copy(
                zbuf.at[pl.ds(0, zrem)],
                agg_s.at[pl.ds(sid * ZROWS, zrem)], rs0).wait()

    pltpu.make_async_copy(ei_hbm.at[0, pl.ds(ebase, EPT)],
                          idx_v.at[pl.ds(0, EPT)], rs1).wait()

    @pl.when(wid < NX)
    def _():
        pltpu.sync_copy(ei_hbm.at[0, pl.ds(NW * EPT + wid * XCH, XCH)],
                        idx_v.at[pl.ds(EPT, XCH)])

    plsc.subcore_barrier()

    buf0 = ebuf.at[0]
    buf1 = ebuf.at[1]
    pltpu.async_copy(ea_hbm.at[pl.ds(ebase, CH)], buf0, rs0)

    def step(i, carry):
        j0 = 2 * i
        j1 = 2 * i + 1
        pltpu.async_copy(ea_hbm.at[pl.ds(ebase + j1 * CH, CH)], buf1, rs1)
        pltpu.make_async_copy(ea_hbm.at[pl.ds(ebase, CH)], buf0, rs0).wait()
        pltpu.sync_copy(buf0, agg_s.at[idx_v.at[pl.ds(j0 * CH, CH)]],
                        add=True)
        j2 = jnp.minimum(j0 + 2, CPT - 1)
        pltpu.async_copy(ea_hbm.at[pl.ds(ebase + j2 * CH, CH)], buf0, rs0)
        pltpu.make_async_copy(ea_hbm.at[pl.ds(ebase, CH)], buf1, rs1).wait()
        pltpu.sync_copy(buf1, agg_s.at[idx_v.at[pl.ds(j1 * CH, CH)]],
                        add=True)
        return carry

    lax.fori_loop(0, CPT // 2, step, 0)
    pltpu.make_async_copy(ea_hbm.at[pl.ds(ebase, CH)], buf0, rs0).wait()

    @pl.when(wid < NX)
    def _():
        pltpu.sync_copy(ea_hbm.at[pl.ds(NW * EPT + wid * XCH, XCH)], buf0)
        pltpu.sync_copy(buf0, agg_s.at[idx_v.at[pl.ds(EPT, XCH)]], add=True)

    plsc.subcore_barrier()

    @pl.when(sid < ZT)
    def _():
        pltpu.sync_copy(agg_s.at[pl.ds(sid * ZROWS, ZROWS)],
                        out_hbm.at[cid, pl.ds(sid * ZROWS, ZROWS)])


BN = 5000


def _mlp_a_body(x_ref, w1x_ref, b1_ref, t_ref):
    t_ref[...] = (jnp.dot(x_ref[...], w1x_ref[...],
                          preferred_element_type=jnp.float32) + b1_ref[...])


def _mlp_a(x, w1x, b1):
    return pl.pallas_call(
        _mlp_a_body,
        grid=(N_NODES // BN,),
        in_specs=[
            pl.BlockSpec((BN, D), lambda i: (i, 0)),
            pl.BlockSpec((D, D), lambda i: (0, 0)),
            pl.BlockSpec((1, D), lambda i: (0, 0)),
        ],
        out_specs=pl.BlockSpec((BN, D), lambda i: (i, 0)),
        out_shape=jax.ShapeDtypeStruct((N_NODES, D), jnp.float32),
    )(x, w1x, b1)


def _mlp_b_body(t_ref, p_ref, w1a_ref, w2_ref, b2_ref, o_ref):
    agg = p_ref[0] + p_ref[1]
    h = t_ref[...] + jnp.dot(agg, w1a_ref[...],
                             preferred_element_type=jnp.float32)
    h = jnp.maximum(h, 0.0)
    o_ref[...] = (jnp.dot(h, w2_ref[...], preferred_element_type=jnp.float32)
                  + b2_ref[...])


def _mlp_b(t, parts, w1a, w2, b2):
    return pl.pallas_call(
        _mlp_b_body,
        grid=(N_NODES // BN,),
        in_specs=[
            pl.BlockSpec((BN, D), lambda i: (i, 0)),
            pl.BlockSpec((NC, BN, D), lambda i: (0, i, 0)),
            pl.BlockSpec((D, D), lambda i: (0, 0)),
            pl.BlockSpec((D, D), lambda i: (0, 0)),
            pl.BlockSpec((1, D), lambda i: (0, 0)),
        ],
        out_specs=pl.BlockSpec((BN, D), lambda i: (i, 0)),
        out_shape=jax.ShapeDtypeStruct((N_NODES, D), jnp.float32),
    )(t, parts, w1a, w2, b2)


def kernel(x, edge_index, edge_attr, u, batch, W1, b1, W2, b2):
    parts = _scatter_sc(edge_index.astype(jnp.int32), edge_attr)
    w1T = W1.T
    t = _mlp_a(x, w1T[:D], b1.reshape(1, D))
    return _mlp_b(t, parts, w1T[D:], W2.T, b2.reshape(1, D))
